# Initial kernel scaffold; baseline (speedup 1.0000x reference)
import jax
import jax.numpy as jnp
from jax.experimental import pallas as pl


def kernel(x):
    raise NotImplementedError("write your pallas kernel here")


# speedup vs baseline: 1.3001x; 1.3001x over previous
import functools

import jax
import jax.numpy as jnp
from jax import lax
from jax.experimental import pallas as pl
from jax.experimental.pallas import tpu as pltpu
from jax.experimental.pallas import tpu_sc as plsc

B = 16384
F = 26
FP = 32
C = 100
ROW = F * C
BR = 16


@functools.lru_cache(maxsize=1)
def _build():
    info = plsc.get_sparse_core_info()
    nw = info.num_cores * info.num_subcores
    rows_w = B // nw
    nb = rows_w // BR

    mesh = plsc.VectorSubcoreMesh(core_axis_name="c", subcore_axis_name="s")

    @functools.partial(
        pl.kernel,
        out_type=jax.ShapeDtypeStruct((B * ROW,), jnp.int32),
        mesh=mesh,
        compiler_params=pltpu.CompilerParams(needs_layout_passes=False),
        scratch_types=[
            pltpu.VMEM((rows_w * FP,), jnp.int32),
            pltpu.VMEM((BR * ROW + 512,), jnp.int32),
        ],
    )
    def onehot(x_hbm, out_hbm, xv, buf):
        wid = lax.axis_index("s") * info.num_cores + lax.axis_index("c")
        base = wid * rows_w

        i16 = lax.broadcasted_iota(jnp.int32, (16,), 0)
        ca = i16 * C
        cb = (i16 + 16) * C
        mb = i16 < (F - 16)
        ones = jnp.ones((16,), jnp.int32)
        zeros = jnp.zeros((16,), jnp.int32)

        pltpu.sync_copy(x_hbm.at[pl.ds(base * FP, rows_w * FP)], xv)

        def zbody(i, _):
            for u in range(4):
                buf[pl.ds(i * 64 + u * 16, 16)] = zeros
            return 0

        lax.fori_loop(0, (BR * ROW) // 64, zbody, 0)

        def block(g, _):
            for r in range(BR):
                off = (g * BR + r) * FP
                xa = xv[pl.ds(off, 16)]
                xb = xv[pl.ds(off + 16, 16)]
                plsc.store_scatter(buf, [xa + (ca + r * ROW)], ones)
                plsc.store_scatter(buf, [xb + (cb + r * ROW)], ones, mask=mb)
            pltpu.sync_copy(
                buf.at[pl.ds(0, BR * ROW)],
                out_hbm.at[pl.ds((base + g * BR) * ROW, BR * ROW)])
            for r in range(BR):
                off = (g * BR + r) * FP
                xa = xv[pl.ds(off, 16)]
                xb = xv[pl.ds(off + 16, 16)]
                plsc.store_scatter(buf, [xa + (ca + r * ROW)], zeros)
                plsc.store_scatter(buf, [xb + (cb + r * ROW)], zeros, mask=mb)
            return 0

        lax.fori_loop(0, nb, block, 0)

    return onehot


def kernel(x):
    xp = jnp.pad(x, ((0, 0), (0, FP - F)))
    out = _build()(xp.reshape(-1))
    return out.reshape(B, ROW)


# speedup vs baseline: 1.3086x; 1.0065x over previous
import functools

import jax
import jax.numpy as jnp
from jax import lax
from jax.experimental import pallas as pl
from jax.experimental.pallas import tpu as pltpu
from jax.experimental.pallas import tpu_sc as plsc

B = 16384
F = 26
FP = 32
C = 100
ROW = F * C
BR = 16
BLK = BR * ROW


@functools.lru_cache(maxsize=1)
def _build():
    info = plsc.get_sparse_core_info()
    nw = info.num_cores * info.num_subcores
    rows_w = B // nw
    nb = rows_w // BR

    mesh = plsc.VectorSubcoreMesh(core_axis_name="c", subcore_axis_name="s")

    @functools.partial(
        pl.kernel,
        out_type=jax.ShapeDtypeStruct((B * ROW,), jnp.int32),
        mesh=mesh,
        compiler_params=pltpu.CompilerParams(needs_layout_passes=False),
        scratch_types=[
            pltpu.VMEM((rows_w * FP,), jnp.int32),
            pltpu.VMEM((BLK + 512,), jnp.int32),
            pltpu.VMEM((BLK + 512,), jnp.int32),
            pltpu.SemaphoreType.DMA,
            pltpu.SemaphoreType.DMA,
        ],
    )
    def onehot(x_hbm, out_hbm, xv, buf0, buf1, sem0, sem1):
        wid = lax.axis_index("s") * info.num_cores + lax.axis_index("c")
        base = wid * rows_w

        i16 = lax.broadcasted_iota(jnp.int32, (16,), 0)
        ca = i16 * C
        cb = (i16 + 16) * C
        mb = i16 < (F - 16)
        ones = jnp.ones((16,), jnp.int32)
        zeros = jnp.zeros((16,), jnp.int32)

        pltpu.sync_copy(x_hbm.at[pl.ds(base * FP, rows_w * FP)], xv)

        def scat(g, buf, val):
            for r in range(BR):
                off = (g * BR + r) * FP
                xa = xv[pl.ds(off, 16)]
                xb = xv[pl.ds(off + 16, 16)]
                plsc.store_scatter(buf, [xa + (ca + r * ROW)], val)
                plsc.store_scatter(buf, [xb + (cb + r * ROW)], val, mask=mb)

        def dma(g, buf, sem):
            return pltpu.make_async_copy(
                buf.at[pl.ds(0, BLK)],
                out_hbm.at[pl.ds((base + g * BR) * ROW, BLK)],
                sem)

        def zbody(i, _):
            for u in range(4):
                s = i * 64 + u * 16
                buf0[pl.ds(s, 16)] = zeros
                buf1[pl.ds(s, 16)] = zeros
            return 0

        lax.fori_loop(0, BLK // 64, zbody, 0)

        scat(0, buf0, ones)
        dma(0, buf0, sem0).start()
        scat(1, buf1, ones)
        dma(1, buf1, sem1).start()

        def step(k, _):
            g = 2 + 2 * k
            for b, buf, sem in ((0, buf0, sem0), (1, buf1, sem1)):
                gg = g + b
                dma(gg - 2, buf, sem).wait()
                scat(gg - 2, buf, zeros)
                scat(gg, buf, ones)
                dma(gg, buf, sem).start()
            return 0

        lax.fori_loop(0, (nb - 2) // 2, step, 0)

        dma(nb - 2, buf0, sem0).wait()
        dma(nb - 1, buf1, sem1).wait()

    return onehot


def kernel(x):
    xp = jnp.pad(x, ((0, 0), (0, FP - F)))
    out = _build()(xp.reshape(-1))
    return out.reshape(B, ROW)
